# baseline (speedup 1.0000x reference)
import jax
import jax.numpy as jnp
from jax import lax
from jax.experimental import pallas as pl
from jax.experimental.pallas import tpu as pltpu

_NZ = 3
_TOKEN_TILE = 512
_NBUF = 3


def _router_body(pos_ref, zb_ref, zt_ref, h_hbm, wt_ref, b_ref,
                 vals_ref, idx_ref, hbuf, sems):
    i = pl.program_id(0)
    nt = pl.num_programs(0)
    T = hbuf.shape[1]
    E = wt_ref.shape[1] // _NZ

    H2 = T // 2

    def start(step):
        slot = lax.rem(step, _NBUF)
        pltpu.make_async_copy(
            h_hbm.at[pl.ds(step * T, H2), :],
            hbuf.at[slot, pl.ds(0, H2)], sems.at[slot, 0],
        ).start()
        pltpu.make_async_copy(
            h_hbm.at[pl.ds(step * T + H2, H2), :],
            hbuf.at[slot, pl.ds(H2, H2)], sems.at[slot, 1],
        ).start()

    @pl.when(i == 0)
    def _():
        for j in range(_NBUF - 1):
            start(j)

    @pl.when(i + _NBUF - 1 < nt)
    def _():
        start(i + _NBUF - 1)

    slot = lax.rem(i, _NBUF)
    pltpu.make_async_copy(
        h_hbm.at[pl.ds(i * T, H2), :],
        hbuf.at[slot, pl.ds(0, H2)], sems.at[slot, 0],
    ).wait()
    pltpu.make_async_copy(
        h_hbm.at[pl.ds(i * T + H2, H2), :],
        hbuf.at[slot, pl.ds(H2, H2)], sems.at[slot, 1],
    ).wait()

    acc = jnp.dot(hbuf[slot], wt_ref[...], preferred_element_type=jnp.float32)
    logits3 = acc + b_ref[...]

    pos = pos_ref[...]
    zt = zt_ref[0]
    zws = []
    for z in range(_NZ):
        left = zb_ref[z]
        right = zb_ref[z + 1]
        zw = jax.nn.sigmoid(zt * (pos - left)) * jax.nn.sigmoid(zt * (right - pos))
        zws.append(zw)
    zsum = jnp.maximum(zws[0] + zws[1] + zws[2], 1e-8)

    comb = (zws[0] / zsum) * logits3[:, 0:E]
    comb = comb + (zws[1] / zsum) * logits3[:, E:2 * E]
    comb = comb + (zws[2] / zsum) * logits3[:, 2 * E:3 * E]

    m = jnp.max(comb, axis=1, keepdims=True)
    e = jnp.exp(comb - m)
    s = jnp.sum(e, axis=1, keepdims=True)

    iota = lax.broadcasted_iota(jnp.int32, comb.shape, 1)
    i1 = jnp.min(jnp.where(comb == m, iota, E), axis=1, keepdims=True)
    pm = jnp.where(iota == i1, -jnp.inf, comb)
    m2 = jnp.max(pm, axis=1, keepdims=True)
    i2 = jnp.min(jnp.where(pm == m2, iota, E), axis=1, keepdims=True)
    v1 = 1.0 / s
    v2 = jnp.exp(m2 - m) / s

    vals_ref[...] = jnp.concatenate([v1, v2], axis=1)
    idx_ref[...] = jnp.concatenate([i1, i2], axis=1)


def kernel(hidden_states, positions, zone_boundaries, W0, W1, W2, b0, b1, b2, zone_temp):
    Bb, Ss, Dd = hidden_states.shape
    E = W0.shape[0]
    if positions.ndim == 1:
        positions = jnp.broadcast_to(positions[None, :], (Bb, Ss))
    BS = Bb * Ss
    posf = positions.astype(jnp.float32).reshape(BS, 1)
    h = hidden_states.reshape(BS, Dd)
    wt = jnp.concatenate([W0, W1, W2], axis=0).T
    bstack = jnp.concatenate([b0, b1, b2], axis=0).reshape(1, _NZ * E)
    zb = zone_boundaries.astype(jnp.float32)
    zt = jnp.reshape(zone_temp.astype(jnp.float32), (1,))

    T = _TOKEN_TILE
    grid = (BS // T,)
    vals, idx = pl.pallas_call(
        _router_body,
        grid=grid,
        in_specs=[
            pl.BlockSpec((T, 1), lambda i: (i, 0)),
            pl.BlockSpec(memory_space=pltpu.SMEM),
            pl.BlockSpec(memory_space=pltpu.SMEM),
            pl.BlockSpec(memory_space=pl.ANY),
            pl.BlockSpec((Dd, _NZ * E), lambda i: (0, 0)),
            pl.BlockSpec((1, _NZ * E), lambda i: (0, 0)),
        ],
        out_specs=[
            pl.BlockSpec((T, 2), lambda i: (i, 0)),
            pl.BlockSpec((T, 2), lambda i: (i, 0)),
        ],
        out_shape=[
            jax.ShapeDtypeStruct((BS, 2), jnp.float32),
            jax.ShapeDtypeStruct((BS, 2), jnp.int32),
        ],
        scratch_shapes=[
            pltpu.VMEM((_NBUF, T, Dd), jnp.float32),
            pltpu.SemaphoreType.DMA((_NBUF, 2)),
        ],
        compiler_params=pltpu.CompilerParams(
            dimension_semantics=("arbitrary",),
        ),
    )(posf, zb, zt, h, wt, bstack)
    return vals.reshape(Bb, Ss, 2), idx.reshape(Bb, Ss, 2)


# speedup vs baseline: 1.0057x; 1.0057x over previous
import jax
import jax.numpy as jnp
from jax import lax
from jax.experimental import pallas as pl
from jax.experimental.pallas import tpu as pltpu

_NZ = 3
_TOKEN_TILE = 512
_NBUF = 3


def _router_body(pos_ref, zb_ref, zt_ref, h_hbm, wt_ref, b_ref,
                 vals_ref, idx_ref, hbuf, sems):
    i = pl.program_id(0)
    nt = pl.num_programs(0)
    T = hbuf.shape[1]
    E = wt_ref.shape[1] // _NZ

    def start(step):
        slot = lax.rem(step, _NBUF)
        pltpu.make_async_copy(
            h_hbm.at[pl.ds(step * T, T), :], hbuf.at[slot], sems.at[slot]
        ).start()

    @pl.when(i == 0)
    def _():
        for j in range(_NBUF - 1):
            start(j)

    @pl.when(i + _NBUF - 1 < nt)
    def _():
        start(i + _NBUF - 1)

    slot = lax.rem(i, _NBUF)
    pltpu.make_async_copy(
        h_hbm.at[pl.ds(i * T, T), :], hbuf.at[slot], sems.at[slot]
    ).wait()

    acc = jnp.dot(hbuf[slot], wt_ref[...], preferred_element_type=jnp.float32)
    logits3 = acc + b_ref[...]

    pos = pos_ref[...]
    zt = zt_ref[0]
    zws = []
    for z in range(_NZ):
        left = zb_ref[z]
        right = zb_ref[z + 1]
        zw = jax.nn.sigmoid(zt * (pos - left)) * jax.nn.sigmoid(zt * (right - pos))
        zws.append(zw)
    zsum = jnp.maximum(zws[0] + zws[1] + zws[2], 1e-8)

    comb = (zws[0] / zsum) * logits3[:, 0:E]
    comb = comb + (zws[1] / zsum) * logits3[:, E:2 * E]
    comb = comb + (zws[2] / zsum) * logits3[:, 2 * E:3 * E]

    m = jnp.max(comb, axis=1, keepdims=True)
    e = jnp.exp(comb - m)
    s = jnp.sum(e, axis=1, keepdims=True)

    iota = lax.broadcasted_iota(jnp.int32, comb.shape, 1)
    i1 = jnp.min(jnp.where(comb == m, iota, E), axis=1, keepdims=True)
    pm = jnp.where(iota == i1, -jnp.inf, comb)
    m2 = jnp.max(pm, axis=1, keepdims=True)
    i2 = jnp.min(jnp.where(pm == m2, iota, E), axis=1, keepdims=True)
    v1 = 1.0 / s
    v2 = jnp.exp(m2 - m) / s

    vals_ref[...] = jnp.concatenate([v1, v2], axis=1)
    idx_ref[...] = jnp.concatenate([i1, i2], axis=1)


def kernel(hidden_states, positions, zone_boundaries, W0, W1, W2, b0, b1, b2, zone_temp):
    Bb, Ss, Dd = hidden_states.shape
    E = W0.shape[0]
    if positions.ndim == 1:
        positions = jnp.broadcast_to(positions[None, :], (Bb, Ss))
    BS = Bb * Ss
    posf = positions.astype(jnp.float32).reshape(BS, 1)
    h = hidden_states.reshape(BS, Dd)
    wt = jnp.concatenate([W0, W1, W2], axis=0).T
    bstack = jnp.concatenate([b0, b1, b2], axis=0).reshape(1, _NZ * E)
    zb = zone_boundaries.astype(jnp.float32)
    zt = jnp.reshape(zone_temp.astype(jnp.float32), (1,))

    T = _TOKEN_TILE
    grid = (BS // T,)
    vals, idx = pl.pallas_call(
        _router_body,
        grid=grid,
        in_specs=[
            pl.BlockSpec((T, 1), lambda i: (i, 0)),
            pl.BlockSpec(memory_space=pltpu.SMEM),
            pl.BlockSpec(memory_space=pltpu.SMEM),
            pl.BlockSpec(memory_space=pl.ANY),
            pl.BlockSpec((Dd, _NZ * E), lambda i: (0, 0)),
            pl.BlockSpec((1, _NZ * E), lambda i: (0, 0)),
        ],
        out_specs=[
            pl.BlockSpec((T, 2), lambda i: (i, 0)),
            pl.BlockSpec((T, 2), lambda i: (i, 0)),
        ],
        out_shape=[
            jax.ShapeDtypeStruct((BS, 2), jnp.float32),
            jax.ShapeDtypeStruct((BS, 2), jnp.int32),
        ],
        scratch_shapes=[
            pltpu.VMEM((_NBUF, T, Dd), jnp.float32),
            pltpu.SemaphoreType.DMA((_NBUF,)),
        ],
        compiler_params=pltpu.CompilerParams(
            dimension_semantics=("arbitrary",),
        ),
    )(posf, zb, zt, h, wt, bstack)
    return vals.reshape(Bb, Ss, 2), idx.reshape(Bb, Ss, 2)
